# baseline (speedup 1.0000x reference)
import functools

import jax
import jax.numpy as jnp
from jax import lax
from jax.experimental import pallas as pl
from jax.experimental.pallas import tpu as pltpu
from jax.experimental.pallas import tpu_sc as plsc

N = 16384
C = 1000
L = 16

_info = plsc.get_sparse_core_info()
_NC, _NS = _info.num_cores, _info.num_subcores
_NW = _NC * _NS
_PER_W = N // _NW
_GW = 128
_NG = _PER_W // _GW


def _focal_kernel(inpt_hbm, tgt_hbm, alpha_hbm, out_hbm,
                  tgt_v, acc_v, patch_v, alpha_v,
                  s0, s1, s2, s3):
    sems = (s0, s1, s2, s3)
    wid = lax.axis_index("s") * _NC + lax.axis_index("c")
    base = wid * _PER_W

    pltpu.sync_copy(tgt_hbm.at[pl.ds(base, _PER_W)], tgt_v)
    pltpu.sync_copy(alpha_hbm, alpha_v)

    copies = []
    for g in range(_NG):
        i0 = base + g * _GW
        copies.append((
            pltpu.async_copy(
                inpt_hbm.at[tgt_v.at[pl.ds(g * _GW, _GW)], pl.ds(i0, _GW)],
                patch_v.at[pl.ds(g * _GW, _GW), :],
                sems[g]),
        ))

    lane = lax.iota(jnp.int32, L)
    acc = jnp.zeros((L,), jnp.float32)
    for g in range(_NG):
        for cp in copies[g]:
            cp.wait()
        def row_body(k16, acc):
            xv = jnp.zeros((L,), jnp.float32)
            for k in range(L):
                v = patch_v[g * _GW + k16 * L + k, pl.ds(k16 * L, L)]
                xv = jnp.where(lane == k, v, xv)
            p = jnp.exp(xv)
            q = 1.0 - p
            w = q * q * xv
            tvec = tgt_v[pl.ds(g * _GW + k16 * L, L)]
            for k in range(L):
                t = tvec[k]
                al = (t // L) * L
                aw = alpha_v[pl.ds(al, L)]
                acc = acc + jnp.where(lane == t - al, aw * w[k], 0.0)
            return acc
        acc = lax.fori_loop(0, _GW // L, row_body, acc)
        for k16 in range(0):
            pass
    acc_v[...] = acc
    pltpu.sync_copy(acc_v, out_hbm.at[pl.ds(wid * L, L)])


@jax.jit
def _focal_call(inp_t, tgt, alpha_flat):
    mesh = plsc.VectorSubcoreMesh(core_axis_name="c", subcore_axis_name="s")
    kern = functools.partial(
        pl.kernel,
        mesh=mesh,
        out_type=jax.ShapeDtypeStruct((_NW * L,), jnp.float32),
        scratch_types=(
            [pltpu.VMEM((_PER_W,), jnp.int32),
             pltpu.VMEM((L,), jnp.float32),
             pltpu.VMEM((_PER_W, _GW), jnp.float32)]
            + [pltpu.VMEM((C,), jnp.float32)]
            + [pltpu.SemaphoreType.DMA for _ in range(_NG)]
        ),
    )(_focal_kernel)
    partials = kern(inp_t, tgt, alpha_flat)
    return -(jnp.sum(partials) / jnp.float32(N))


def kernel(inputs, targets, alpha):
    tgt = targets.astype(jnp.int32)
    alpha_flat = alpha.reshape(-1).astype(jnp.float32)
    return _focal_call(inputs.T, tgt, alpha_flat)


# speedup vs baseline: 1.1013x; 1.1013x over previous
import functools

import jax
import jax.numpy as jnp
from jax import lax
from jax.experimental import pallas as pl
from jax.experimental.pallas import tpu as pltpu
from jax.experimental.pallas import tpu_sc as plsc

N = 16384
C = 1000
L = 16

_info = plsc.get_sparse_core_info()
_NC, _NS = _info.num_cores, _info.num_subcores
_NW = _NC * _NS
_PER_W = N // _NW
_GW = 128
_NG = _PER_W // _GW


def _focal_kernel(inpt_hbm, tgt_hbm, alpha_hbm, out_hbm,
                  tgt_v, acc_v, patch_v, alpha_v,
                  s0, s1, s2, s3):
    sems = (s0, s1, s2, s3)
    wid = lax.axis_index("s") * _NC + lax.axis_index("c")
    base = wid * _PER_W

    pltpu.sync_copy(tgt_hbm.at[pl.ds(base, _PER_W)], tgt_v)
    pltpu.sync_copy(alpha_hbm, alpha_v.at[pl.ds(0, C)])

    copies = []
    for g in range(_NG):
        i0 = base + g * _GW
        copies.append((
            pltpu.async_copy(
                inpt_hbm.at[tgt_v.at[pl.ds(g * _GW, _GW)], pl.ds(i0, _GW)],
                patch_v.at[pl.ds(g * _GW, _GW), :],
                sems[g]),
        ))

    lane = lax.iota(jnp.int32, L)
    acc = jnp.zeros((L,), jnp.float32)
    for g in range(_NG):
        for cp in copies[g]:
            cp.wait()
        def row_body(k16, acc):
            xv = jnp.zeros((L,), jnp.float32)
            for k in range(L):
                v = patch_v[g * _GW + k16 * L + k, pl.ds(k16 * L, L)]
                xv = jnp.where(lane == k, v, xv)
            p = jnp.exp(xv)
            q = 1.0 - p
            w = q * q * xv
            tvec = tgt_v[pl.ds(g * _GW + k16 * L, L)]
            for k in range(L):
                t = tvec[k]
                aw = alpha_v[pl.ds(t, L)]
                acc = acc + jnp.where(lane == 0, aw * w[k], 0.0)
            return acc
        acc = lax.fori_loop(0, _GW // L, row_body, acc)
        for k16 in range(0):
            pass
    acc_v[...] = acc
    pltpu.sync_copy(acc_v, out_hbm.at[pl.ds(wid * L, L)])


@jax.jit
def _focal_call(inp_t, tgt, alpha_flat):
    mesh = plsc.VectorSubcoreMesh(core_axis_name="c", subcore_axis_name="s")
    kern = functools.partial(
        pl.kernel,
        mesh=mesh,
        out_type=jax.ShapeDtypeStruct((_NW * L,), jnp.float32),
        scratch_types=(
            [pltpu.VMEM((_PER_W,), jnp.int32),
             pltpu.VMEM((L,), jnp.float32),
             pltpu.VMEM((_PER_W, _GW), jnp.float32)]
            + [pltpu.VMEM((1024,), jnp.float32)]
            + [pltpu.SemaphoreType.DMA for _ in range(_NG)]
        ),
    )(_focal_kernel)
    partials = kern(inp_t, tgt, alpha_flat)
    return -(jnp.sum(partials) / jnp.float32(N))


def kernel(inputs, targets, alpha):
    tgt = targets.astype(jnp.int32)
    alpha_flat = alpha.reshape(-1).astype(jnp.float32)
    return _focal_call(inputs.T, tgt, alpha_flat)


# speedup vs baseline: 1.1239x; 1.0206x over previous
import functools

import jax
import jax.numpy as jnp
from jax import lax
from jax.experimental import pallas as pl
from jax.experimental.pallas import tpu as pltpu
from jax.experimental.pallas import tpu_sc as plsc

N = 16384
C = 1000
L = 16

_info = plsc.get_sparse_core_info()
_NC, _NS = _info.num_cores, _info.num_subcores
_NW = _NC * _NS
_PER_W = N // _NW
_GW = 128
_NG = _PER_W // _GW


def _focal_kernel(inpt_hbm, tgt_hbm, alpha_hbm, out_hbm,
                  tgt_v, acc_v, patch_v, alpha_v,
                  s0, s1, s2, s3):
    sems = (s0, s1, s2, s3)
    wid = lax.axis_index("s") * _NC + lax.axis_index("c")
    base = wid * _PER_W

    pltpu.sync_copy(tgt_hbm.at[pl.ds(base, _PER_W)], tgt_v)
    pltpu.sync_copy(alpha_hbm, alpha_v.at[pl.ds(0, C)])

    copies = []
    for g in range(_NG):
        i0 = base + g * _GW
        copies.append((
            pltpu.async_copy(
                inpt_hbm.at[tgt_v.at[pl.ds(g * _GW, _GW)], pl.ds(i0, _GW)],
                patch_v.at[pl.ds(g * _GW, _GW), :],
                sems[g]),
        ))

    for grp in copies:
        for cp in grp:
            cp.wait()

    lane = lax.iota(jnp.int32, L)

    def row_body(j, acc):
        col = (j * L) % _GW
        xv = jnp.zeros((L,), jnp.float32)
        for k in range(L):
            v = patch_v[j * L + k, pl.ds(col, L)]
            xv = jnp.where(lane == k, v, xv)
        p = jnp.exp(xv)
        q = 1.0 - p
        w = q * q * xv
        tvec = tgt_v[pl.ds(j * L, L)]
        for k in range(L):
            t = tvec[k]
            aw = alpha_v[pl.ds(t, L)]
            acc = acc + jnp.where(lane == 0, aw * w[k], 0.0)
        return acc

    acc = lax.fori_loop(0, _PER_W // L, row_body,
                        jnp.zeros((L,), jnp.float32))
    acc_v[...] = acc
    pltpu.sync_copy(acc_v, out_hbm.at[pl.ds(wid * L, L)])


@jax.jit
def _focal_call(inp_t, tgt, alpha_flat):
    mesh = plsc.VectorSubcoreMesh(core_axis_name="c", subcore_axis_name="s")
    kern = functools.partial(
        pl.kernel,
        mesh=mesh,
        out_type=jax.ShapeDtypeStruct((_NW * L,), jnp.float32),
        scratch_types=(
            [pltpu.VMEM((_PER_W,), jnp.int32),
             pltpu.VMEM((L,), jnp.float32),
             pltpu.VMEM((_PER_W, _GW), jnp.float32)]
            + [pltpu.VMEM((1024,), jnp.float32)]
            + [pltpu.SemaphoreType.DMA for _ in range(_NG)]
        ),
    )(_focal_kernel)
    partials = kern(inp_t, tgt, alpha_flat)
    return -(jnp.sum(partials) / jnp.float32(N))


def kernel(inputs, targets, alpha):
    tgt = targets.astype(jnp.int32)
    alpha_flat = alpha.reshape(-1).astype(jnp.float32)
    return _focal_call(inputs.T, tgt, alpha_flat)
